# baseline (speedup 1.0000x reference)
import jax
import jax.numpy as jnp
from jax import lax
from jax.experimental import pallas as pl
from jax.experimental.pallas import tpu as pltpu
from jax.experimental.pallas import tpu_sc as plsc

VOCAB = 100000
EMB_DIM = 300
NUM_CLASSES = 2
BATCH = 4096
SEQ = 200

DP = 16
NC, NS = 2, 16
NW = NC * NS
ROWS_PER_W = BATCH // NW
LANES = 16
BLOCKS_PER_W = ROWS_PER_W // LANES

PROJ_BLK = 4096
NBLK = (VOCAB + PROJ_BLK - 1) // PROJ_BLK
VOCAB_PAD = NBLK * PROJ_BLK
SUBR = PROJ_BLK // 64


def _proj_body(embT_ref, wt_ref, out_ref):
    res = lax.dot_general(embT_ref[...], wt_ref[...],
                          dimension_numbers=(((0,), (0,)), ((), ())),
                          preferred_element_type=jnp.float32)
    res2 = res[:, :NUM_CLASSES]
    out_ref[...] = jnp.concatenate(
        [res2[SUBR * k:SUBR * (k + 1)] for k in range(64)],
        axis=1).astype(jnp.bfloat16)


def _project(embT, wt):
    return pl.pallas_call(
        _proj_body,
        grid=(NBLK,),
        in_specs=[
            pl.BlockSpec((EMB_DIM, PROJ_BLK), lambda i: (0, i)),
            pl.BlockSpec((EMB_DIM, DP), lambda i: (0, 0)),
        ],
        out_specs=pl.BlockSpec((SUBR, 128), lambda i: (i, 0)),
        out_shape=jax.ShapeDtypeStruct((NBLK * SUBR, 128), jnp.bfloat16),
    )(embT, wt)


def _pool_body(tbl_hbm, x3_hbm, bvec_hbm, out_hbm, tbl_v, idx_v, b_v, out_v,
               sem0, sem1):
    wid = lax.axis_index("s") * NC + lax.axis_index("c")

    tbl_cp = pltpu.async_copy(tbl_hbm, tbl_v, sem0)
    idx_cp = pltpu.async_copy(
        x3_hbm.at[:, pl.ds(wid * ROWS_PER_W, ROWS_PER_W)], idx_v, sem1)
    pltpu.sync_copy(bvec_hbm, b_v)
    idx_cp.wait()
    tbl_cp.wait()

    zero = jnp.zeros((LANES,), jnp.float32)
    hi_mask = jnp.full((LANES,), -65536, jnp.int32)
    sh16 = jnp.full((LANES,), 16, jnp.int32)

    for blk in range(BLOCKS_PER_W):
        def body(j, carry):
            a0, a1 = carry
            v = idx_v[j, pl.ds(blk * LANES, LANES)]
            u = ((v & ~(PROJ_BLK - 1)) | ((v & (SUBR - 1)) << 6)
                 | ((v >> 6) & 63))
            w = plsc.load_gather(tbl_v, [u])
            f0 = plsc.bitcast(lax.shift_left(w, sh16), jnp.float32)
            f1 = plsc.bitcast(lax.bitwise_and(w, hi_mask), jnp.float32)
            return (a0 + f0, a1 + f1)

        z0, z1 = lax.fori_loop(0, SEQ, body, (zero, zero), unroll=25)
        z0 = z0 + b_v[0]
        z1 = z1 + b_v[1]
        out_v[pl.ds(blk * LANES, LANES)] = 1.0 / (1.0 + jnp.exp(-z0))
        out_v[pl.ds(ROWS_PER_W + blk * LANES, LANES)] = (
            1.0 / (1.0 + jnp.exp(-z1)))

    pltpu.sync_copy(out_v, out_hbm.at[wid])


def _pool(tbl, x3, bvec):
    mesh = plsc.VectorSubcoreMesh(core_axis_name="c", subcore_axis_name="s")
    fn = pl.kernel(
        _pool_body,
        mesh=mesh,
        compiler_params=pltpu.CompilerParams(
            use_tc_tiling_on_sc=False, needs_layout_passes=False),
        out_type=jax.ShapeDtypeStruct((NW, NUM_CLASSES * ROWS_PER_W),
                                      jnp.float32),
        scratch_types=[
            pltpu.VMEM((VOCAB_PAD,), jnp.int32),
            pltpu.VMEM((SEQ, ROWS_PER_W), jnp.int32),
            pltpu.VMEM((NUM_CLASSES, LANES), jnp.float32),
            pltpu.VMEM((NUM_CLASSES * ROWS_PER_W,), jnp.float32),
            pltpu.SemaphoreType.DMA,
            pltpu.SemaphoreType.DMA,
        ],
    )
    return fn(tbl, x3, bvec)


def kernel(x, emb, W, b):
    x = x.astype(jnp.int32)
    wt = jnp.zeros((EMB_DIM, DP), jnp.float32).at[:, :NUM_CLASSES].set(
        W.T * (1.0 / SEQ))
    packed = _project(emb.T, wt)
    tbl = lax.bitcast_convert_type(
        packed.reshape(NBLK * SUBR, 64, 2), jnp.int32).reshape(VOCAB_PAD)
    x3 = x.T
    bvec = jnp.tile(b[:, None], (1, LANES)).astype(jnp.float32)
    out = _pool(tbl, x3, bvec)
    return out.reshape(NW, NUM_CLASSES, ROWS_PER_W).transpose(
        0, 2, 1).reshape(BATCH, NUM_CLASSES)


# speedup vs baseline: 1.0294x; 1.0294x over previous
import jax
import jax.numpy as jnp
from jax import lax
from jax.experimental import pallas as pl
from jax.experimental.pallas import tpu as pltpu
from jax.experimental.pallas import tpu_sc as plsc

VOCAB = 100000
EMB_DIM = 300
NUM_CLASSES = 2
BATCH = 4096
SEQ = 200

DP = 16
NC, NS = 2, 16
NW = NC * NS
ROWS_PER_W = BATCH // NW
LANES = 16
BLOCKS_PER_W = ROWS_PER_W // LANES

PROJ_BLK = 4096
NBLK = (VOCAB + PROJ_BLK - 1) // PROJ_BLK
VOCAB_PAD = NBLK * PROJ_BLK
SUBR = PROJ_BLK // 64


def _proj_body(embT_ref, wt_ref, out_ref):
    res = lax.dot_general(embT_ref[...], wt_ref[...],
                          dimension_numbers=(((0,), (0,)), ((), ())),
                          preferred_element_type=jnp.float32)
    res2 = res[:, :NUM_CLASSES]
    out_ref[...] = jnp.concatenate(
        [res2[SUBR * k:SUBR * (k + 1)] for k in range(64)],
        axis=1).astype(jnp.bfloat16)


def _project(embT, wt):
    return pl.pallas_call(
        _proj_body,
        grid=(NBLK,),
        in_specs=[
            pl.BlockSpec((EMB_DIM, PROJ_BLK), lambda i: (0, i)),
            pl.BlockSpec((EMB_DIM, DP), lambda i: (0, 0)),
        ],
        out_specs=pl.BlockSpec((SUBR, 128), lambda i: (i, 0)),
        out_shape=jax.ShapeDtypeStruct((NBLK * SUBR, 128), jnp.bfloat16),
    )(embT, wt)


def _pool_body(tbl_hbm, x3_hbm, bvec_hbm, out_hbm, tbl_v, idx_v, b_v, out_v,
               sem0, sem1):
    wid = lax.axis_index("s") * NC + lax.axis_index("c")

    tbl_cp = pltpu.async_copy(tbl_hbm, tbl_v, sem0)
    idx_cp = pltpu.async_copy(
        x3_hbm.at[:, pl.ds(wid * ROWS_PER_W, ROWS_PER_W)], idx_v, sem1)
    pltpu.sync_copy(bvec_hbm, b_v)
    idx_cp.wait()
    tbl_cp.wait()

    zero = jnp.zeros((LANES,), jnp.float32)
    hi_mask = jnp.full((LANES,), -65536, jnp.int32)
    sh16 = jnp.full((LANES,), 16, jnp.int32)

    for blk in range(BLOCKS_PER_W):
        def body(j, carry):
            a0, a1 = carry
            v = idx_v[j, pl.ds(blk * LANES, LANES)]
            u = ((v & ~(PROJ_BLK - 1)) | ((v & (SUBR - 1)) << 6)
                 | ((v >> 6) & 63))
            w = plsc.load_gather(tbl_v, [u])
            f0 = plsc.bitcast(lax.shift_left(w, sh16), jnp.float32)
            f1 = plsc.bitcast(lax.bitwise_and(w, hi_mask), jnp.float32)
            return (a0 + f0, a1 + f1)

        z0, z1 = lax.fori_loop(0, SEQ, body, (zero, zero), unroll=8)
        z0 = z0 + b_v[0]
        z1 = z1 + b_v[1]
        out_v[pl.ds(blk * LANES, LANES)] = 1.0 / (1.0 + jnp.exp(-z0))
        out_v[pl.ds(ROWS_PER_W + blk * LANES, LANES)] = (
            1.0 / (1.0 + jnp.exp(-z1)))

    pltpu.sync_copy(out_v, out_hbm.at[wid])


def _pool(tbl, x3, bvec):
    mesh = plsc.VectorSubcoreMesh(core_axis_name="c", subcore_axis_name="s")
    fn = pl.kernel(
        _pool_body,
        mesh=mesh,
        compiler_params=pltpu.CompilerParams(
            use_tc_tiling_on_sc=False, needs_layout_passes=False),
        out_type=jax.ShapeDtypeStruct((NW, NUM_CLASSES * ROWS_PER_W),
                                      jnp.float32),
        scratch_types=[
            pltpu.VMEM((VOCAB_PAD,), jnp.int32),
            pltpu.VMEM((SEQ, ROWS_PER_W), jnp.int32),
            pltpu.VMEM((NUM_CLASSES, LANES), jnp.float32),
            pltpu.VMEM((NUM_CLASSES * ROWS_PER_W,), jnp.float32),
            pltpu.SemaphoreType.DMA,
            pltpu.SemaphoreType.DMA,
        ],
    )
    return fn(tbl, x3, bvec)


def kernel(x, emb, W, b):
    x = x.astype(jnp.int32)
    wt = jnp.zeros((EMB_DIM, DP), jnp.float32).at[:, :NUM_CLASSES].set(
        W.T * (1.0 / SEQ))
    packed = _project(emb.T, wt)
    tbl = lax.bitcast_convert_type(
        packed.reshape(NBLK * SUBR, 64, 2), jnp.int32).reshape(VOCAB_PAD)
    x3 = x.T
    bvec = jnp.tile(b[:, None], (1, LANES)).astype(jnp.float32)
    out = _pool(tbl, x3, bvec)
    return out.reshape(NW, NUM_CLASSES, ROWS_PER_W).transpose(
        0, 2, 1).reshape(BATCH, NUM_CLASSES)
